# baseline (speedup 1.0000x reference)
import jax
import jax.numpy as jnp
from jax import lax
from jax.experimental import pallas as pl
from jax.experimental.pallas import tpu as pltpu
from jax.experimental.pallas import tpu_sc as plsc

B = 4096
L = 200
V = 100000
K = 300
DT = 304
TW = 176
NS = 2
CA = 104
CB = L - CA
NW = 32
SPW = B // NW
DJ = TW // 16


def _rne_hi16(u):
  return (u + 0x7FFF + ((u >> 16) & 1)) & jnp.uint32(0xFFFF0000)


def _transform_body(e_ref, w_ref, b_ref, t0_ref, et_ref):
  et_ref[...] = e_ref[...].T
  t = (
      jnp.dot(
          et_ref[...].astype(jnp.bfloat16),
          w_ref[...].astype(jnp.bfloat16),
          preferred_element_type=jnp.float32,
      )
      + b_ref[...]
  )
  u = lax.bitcast_convert_type(t, jnp.uint32)
  a0 = _rne_hi16(u[:, 0:128]) | (_rne_hi16(u[:, 128:256]) >> 16)
  tail = _rne_hi16(u[:, 256:304])
  t0_ref[...] = lax.bitcast_convert_type(
      jnp.concatenate([a0, tail], axis=1), jnp.int32
  )


def _transform_table(embT, w1t_p, b1_p):
  mblk = 4096
  return pl.pallas_call(
      _transform_body,
      grid=(pl.cdiv(V, mblk),),
      in_specs=[
          pl.BlockSpec((K, mblk), lambda i: (0, i)),
          pl.BlockSpec((K, DT), lambda i: (0, 0)),
          pl.BlockSpec((1, DT), lambda i: (0, 0)),
      ],
      out_specs=pl.BlockSpec((mblk, TW), lambda i: (i, 0)),
      out_shape=jax.ShapeDtypeStruct((V, TW), jnp.int32),
      scratch_shapes=[pltpu.VMEM((mblk, K), jnp.float32)],
  )(embT, w1t_p, b1_p)


def _pool_body(t0, x_hbm, w2_hbm, b2_hbm, out_hbm,
               idx_v, r0, w2_v, b2_v, out_v, sem0, sem1):
  wid = lax.axis_index("s") * 2 + lax.axis_index("c")
  ibase = pl.multiple_of(wid * (SPW * L), 8)
  obase = pl.multiple_of(wid * SPW, 8)

  pltpu.sync_copy(x_hbm.at[pl.ds(ibase, SPW * L)], idx_v)
  pltpu.sync_copy(w2_hbm, w2_v)
  pltpu.sync_copy(b2_hbm, b2_v)

  sems = (sem0, sem1)
  cdesc = ((0, CA), (SPW * CA, CB))

  def issue(s, c):
    coff, cn = cdesc[c]
    off = pl.multiple_of(coff + s * cn, 8)
    pltpu.async_copy(
        t0.at[idx_v.at[pl.ds(off, cn)]], r0.at[c, pl.ds(0, cn)], sems[c]
    )

  def wait(c):
    cn = cdesc[c][1]
    pltpu.make_async_copy(
        t0.at[idx_v.at[pl.ds(0, cn)]], r0.at[c, pl.ds(0, cn)], sems[c]
    ).wait()

  issue(0, 0)
  issue(0, 1)

  RUN = 4
  zero32 = jnp.zeros((32,), jnp.bfloat16)
  zero16 = jnp.zeros((16,), jnp.float32)
  lanes = lax.iota(jnp.int32, 16)
  b2f = b2_v[...]

  def body(s, carry):
    vec0, vec1 = carry
    accs = tuple(zero32 for _ in range(DJ))
    for c in range(2):
      wait(c)
      cn = cdesc[c][1]

      def rbody(i, a, c=c):
        a = list(a)
        for dr in range(RUN):
          r = i * RUN + dr
          for j in range(DJ):
            w = r0[c, r, pl.ds(16 * j, 16)]
            a[j] = jnp.maximum(a[j], plsc.bitcast(w, jnp.bfloat16))
        return tuple(a)

      accs = lax.fori_loop(0, cn // RUN, rbody, accs)

      @pl.when(s + 1 < SPW)
      def _():
        issue(s + 1, c)

    pa = [zero16, zero16]
    for j in range(DJ):
      lo, hi = plsc.unpack(accs[j], format=plsc.PackFormat.INTERLEAVED)
      for k in range(2):
        pa[k] = (
            pa[k]
            + lo * w2_v[2 * k, pl.ds(16 * j, 16)]
            + hi * w2_v[2 * k + 1, pl.ds(16 * j, 16)]
        )
    lane = s % 16
    ins = lanes == lane
    vec0 = jnp.where(lane == 0, zero16, vec0)
    vec1 = jnp.where(lane == 0, zero16, vec1)
    vec0 = jnp.where(ins, jnp.sum(pa[0]) + b2f[0], vec0)
    vec1 = jnp.where(ins, jnp.sum(pa[1]) + b2f[1], vec1)

    @pl.when(lane == 15)
    def _():
      goff = pl.multiple_of(s - 15, 8)
      out_v[0, pl.ds(goff, 16)] = vec0
      out_v[1, pl.ds(goff, 16)] = vec1

    return (vec0, vec1)

  lax.fori_loop(0, SPW, body, (zero16, zero16))
  for k in range(2):
    pltpu.sync_copy(out_v.at[k], out_hbm.at[k, pl.ds(obase, SPW)])


def _pool(table, x_flat, w2s, b2v):
  mesh = plsc.VectorSubcoreMesh(
      core_axis_name="c", subcore_axis_name="s", num_cores=2, num_subcores=16
  )
  k = pl.kernel(
      _pool_body,
      out_type=jax.ShapeDtypeStruct((2, B), jnp.float32),
      mesh=mesh,
      scratch_types=[
          pltpu.VMEM((SPW * L,), jnp.int32),
          pltpu.VMEM((2, CA, TW), jnp.int32),
          pltpu.VMEM((4, TW), jnp.float32),
          pltpu.VMEM((16,), jnp.float32),
          pltpu.VMEM((2, SPW), jnp.float32),
          pltpu.SemaphoreType.DMA,
          pltpu.SemaphoreType.DMA,
      ],
      compiler_params=pltpu.CompilerParams(
          use_tc_tiling_on_sc=False, needs_layout_passes=False
      ),
  )
  return k(table, x_flat, w2s, b2v)


@jax.jit
def kernel(x, emb, W1, b1, W2, b2):
  w1t_p = jnp.pad(W1.T, ((0, 0), (0, DT - 300)))
  b1_p = jnp.pad(b1, (0, DT - 300)).reshape(1, DT)

  table = _transform_table(emb.T, w1t_p, b1_p)

  x_i = x.astype(jnp.int32).reshape(NW, SPW, L)
  x_flat = jnp.concatenate(
      [x_i[:, :, :CA].reshape(NW, -1), x_i[:, :, CA:].reshape(NW, -1)], axis=1
  ).reshape(-1)

  w2r = W2.astype(jnp.bfloat16).astype(jnp.float32)
  w2p = jnp.pad(w2r, ((0, 0), (0, 304 - 300)))
  z48 = jnp.zeros((48,), jnp.float32)
  w2s = jnp.stack([
      jnp.concatenate([w2p[0, 128:256], z48]),
      jnp.concatenate([w2p[0, 0:128], w2p[0, 256:304]]),
      jnp.concatenate([w2p[1, 128:256], z48]),
      jnp.concatenate([w2p[1, 0:128], w2p[1, 256:304]]),
  ])
  b2v = jnp.pad(b2, (0, 14))

  out = _pool(table, x_flat, w2s, b2v)
  return out.T


# speedup vs baseline: 1.0701x; 1.0701x over previous
import jax
import jax.numpy as jnp
from jax import lax
from jax.experimental import pallas as pl
from jax.experimental.pallas import tpu as pltpu
from jax.experimental.pallas import tpu_sc as plsc

B = 4096
L = 200
V = 100000
K = 300
DT = 304
DW = 256
NS = 2
CA = 104
CB = L - CA
NW = 32
SPW = B // NW
DJ = DW // 16


def _rne_hi16(u):
  return (u + 0x7FFF + ((u >> 16) & 1)) & jnp.uint32(0xFFFF0000)


def _transform_body(e_ref, w_ref, b_ref, t0_ref, t1_ref, et_ref):
  et_ref[...] = e_ref[...].T
  t = (
      jnp.dot(
          et_ref[...].astype(jnp.bfloat16),
          w_ref[...].astype(jnp.bfloat16),
          preferred_element_type=jnp.float32,
      )
      + b_ref[...]
  )
  u = lax.bitcast_convert_type(t, jnp.uint32)
  a0 = _rne_hi16(u[:, 0:128]) | (_rne_hi16(u[:, 128:256]) >> 16)
  tail = _rne_hi16(u[:, 256:304])
  a1 = jnp.concatenate(
      [tail, jnp.zeros((tail.shape[0], 80), jnp.uint32)], axis=1
  )
  t0_ref[...] = lax.bitcast_convert_type(a0, jnp.int32)
  t1_ref[...] = lax.bitcast_convert_type(a1, jnp.int32)


def _transform_table(embT, w1t_p, b1_p):
  mblk = 8192
  return pl.pallas_call(
      _transform_body,
      grid=(pl.cdiv(V, mblk),),
      in_specs=[
          pl.BlockSpec((K, mblk), lambda i: (0, i)),
          pl.BlockSpec((K, DT), lambda i: (0, 0)),
          pl.BlockSpec((1, DT), lambda i: (0, 0)),
      ],
      out_specs=[
          pl.BlockSpec((mblk, 128), lambda i: (i, 0)) for _ in range(NS)
      ],
      out_shape=[jax.ShapeDtypeStruct((V, 128), jnp.int32) for _ in range(NS)],
      scratch_shapes=[pltpu.VMEM((mblk, K), jnp.float32)],
  )(embT, w1t_p, b1_p)


def _pool_body(t0, t1, x_hbm, w2_hbm, b2_hbm, out_hbm,
               idx_v, r0, r1, w2_v, b2_v, out_v, sem0, sem1):
  wid = lax.axis_index("s") * 2 + lax.axis_index("c")
  ibase = pl.multiple_of(wid * (SPW * L), 8)
  obase = pl.multiple_of(wid * SPW, 8)
  tabs = (t0, t1)
  rows = (r0, r1)

  pltpu.sync_copy(x_hbm.at[pl.ds(ibase, SPW * L)], idx_v)
  pltpu.sync_copy(w2_hbm, w2_v)
  pltpu.sync_copy(b2_hbm, b2_v)

  sems = (sem0, sem1)
  cdesc = ((0, CA), (SPW * CA, CB))

  def issue(s, c):
    coff, cn = cdesc[c]
    off = pl.multiple_of(coff + s * cn, 8)
    for k in range(NS):
      pltpu.async_copy(
          tabs[k].at[idx_v.at[pl.ds(off, cn)]],
          rows[k].at[c, pl.ds(0, cn)],
          sems[c],
      )

  def wait(c):
    cn = cdesc[c][1]
    for k in range(NS):
      pltpu.make_async_copy(
          tabs[k].at[idx_v.at[pl.ds(0, cn)]],
          rows[k].at[c, pl.ds(0, cn)],
          sems[c],
      ).wait()

  issue(0, 0)
  issue(0, 1)

  RUN = 4
  zero32 = jnp.zeros((32,), jnp.bfloat16)
  zero16 = jnp.zeros((16,), jnp.float32)
  lanes = lax.iota(jnp.int32, 16)
  b2f = b2_v[...]

  def body(s, carry):
    vec0, vec1 = carry
    accs = tuple(zero32 for _ in range(DJ))
    for c in range(2):
      wait(c)
      cn = cdesc[c][1]

      def rbody(i, a, c=c):
        a = list(a)
        for dr in range(RUN):
          r = i * RUN + dr
          for j in range(DJ):
            w = rows[j // 8][c, r, pl.ds(16 * (j % 8), 16)]
            a[j] = jnp.maximum(a[j], plsc.bitcast(w, jnp.bfloat16))
        return tuple(a)

      accs = lax.fori_loop(0, cn // RUN, rbody, accs)

      @pl.when(s + 1 < SPW)
      def _():
        issue(s + 1, c)

    pa = [zero16, zero16]
    for j in range(DJ):
      lo, hi = plsc.unpack(accs[j], format=plsc.PackFormat.INTERLEAVED)
      for k in range(2):
        pa[k] = (
            pa[k]
            + lo * w2_v[2 * k, pl.ds(16 * j, 16)]
            + hi * w2_v[2 * k + 1, pl.ds(16 * j, 16)]
        )
    lane = s % 16
    ins = lanes == lane
    vec0 = jnp.where(lane == 0, zero16, vec0)
    vec1 = jnp.where(lane == 0, zero16, vec1)
    vec0 = jnp.where(ins, jnp.sum(pa[0]) + b2f[0], vec0)
    vec1 = jnp.where(ins, jnp.sum(pa[1]) + b2f[1], vec1)

    @pl.when(lane == 15)
    def _():
      goff = pl.multiple_of(s - 15, 8)
      out_v[0, pl.ds(goff, 16)] = vec0
      out_v[1, pl.ds(goff, 16)] = vec1

    return (vec0, vec1)

  lax.fori_loop(0, SPW, body, (zero16, zero16))
  for k in range(2):
    pltpu.sync_copy(out_v.at[k], out_hbm.at[k, pl.ds(obase, SPW)])


def _pool(tables, x_flat, w2s, b2v):
  mesh = plsc.VectorSubcoreMesh(
      core_axis_name="c", subcore_axis_name="s", num_cores=2, num_subcores=16
  )
  k = pl.kernel(
      _pool_body,
      out_type=jax.ShapeDtypeStruct((2, B), jnp.float32),
      mesh=mesh,
      scratch_types=[
          pltpu.VMEM((SPW * L,), jnp.int32),
          pltpu.VMEM((2, CA, 128), jnp.int32),
          pltpu.VMEM((2, CA, 128), jnp.int32),
          pltpu.VMEM((4, DW), jnp.float32),
          pltpu.VMEM((16,), jnp.float32),
          pltpu.VMEM((2, SPW), jnp.float32),
          pltpu.SemaphoreType.DMA,
          pltpu.SemaphoreType.DMA,
      ],
      compiler_params=pltpu.CompilerParams(
          use_tc_tiling_on_sc=False, needs_layout_passes=False
      ),
  )
  return k(tables[0], tables[1], x_flat, w2s, b2v)


@jax.jit
def kernel(x, emb, W1, b1, W2, b2):
  w1t_p = jnp.pad(W1.T, ((0, 0), (0, DT - 300)))
  b1_p = jnp.pad(b1, (0, DT - 300)).reshape(1, DT)

  tables = _transform_table(emb.T, w1t_p, b1_p)

  x_i = x.astype(jnp.int32).reshape(NW, SPW, L)
  x_flat = jnp.concatenate(
      [x_i[:, :, :CA].reshape(NW, -1), x_i[:, :, CA:].reshape(NW, -1)], axis=1
  ).reshape(-1)

  w2r = W2.astype(jnp.bfloat16).astype(jnp.float32)
  w2p = jnp.pad(w2r, ((0, 0), (0, 384 - 300)))
  z128 = jnp.zeros((128,), jnp.float32)
  w2s = jnp.stack([
      jnp.concatenate([w2p[0, 128:256], z128]),
      jnp.concatenate([w2p[0, 0:128], w2p[0, 256:384]]),
      jnp.concatenate([w2p[1, 128:256], z128]),
      jnp.concatenate([w2p[1, 0:128], w2p[1, 256:384]]),
  ])
  b2v = jnp.pad(b2, (0, 14))

  out = _pool(tables, x_flat, w2s, b2v)
  return out.T
